# Initial kernel scaffold; baseline (speedup 1.0000x reference)
import jax
import jax.numpy as jnp
from jax.experimental import pallas as pl


def kernel(white_piece_idx, black_piece_idx, white_castle_idx, black_castle_idx, white_ep_idx, black_ep_idx, white_fifty_idx, black_fifty_idx, W_white_piece, W_black_piece, W_white_castle, W_black_castle, W_white_ep, W_black_ep, W_white_fifty, W_black_fifty):
    raise NotImplementedError("write your pallas kernel here")


# speedup vs baseline: 419.0373x; 419.0373x over previous
import jax
import jax.numpy as jnp
from jax.experimental import pallas as pl
from jax.experimental.pallas import tpu as pltpu

_D = 32
_TB = 1024
_K = 1568


def _body(wpi_ref, bpi_ref, sm_ref, w_ref, out_ref):
    tb = wpi_ref.shape[0]
    wpi2 = jnp.concatenate([wpi_ref[...], wpi_ref[...]], axis=1)
    bpi2 = jnp.concatenate([bpi_ref[...], bpi_ref[...]], axis=1)
    hi = jax.lax.broadcasted_iota(jnp.int32, (tb, 128), 1) // 64
    pieces = []
    for pp in range(6):
        pieces.append((wpi2 == (2 * pp + hi)).astype(jnp.bfloat16))
    for pp in range(6):
        pieces.append((bpi2 == (2 * pp + hi)).astype(jnp.bfloat16))
    c = jax.lax.broadcasted_iota(jnp.int32, (tb, 32), 1)
    segbase = jnp.where(
        c < 4, 0, jnp.where(c < 8, 4, jnp.where(c < 16, 8, jnp.where(
            c < 24, 16, jnp.where(c < 26, 24, jnp.where(c < 28, 26, 100))))))
    pieces.append((sm_ref[...] == (c - segbase)).astype(jnp.bfloat16))
    masks = jnp.concatenate(pieces, axis=1)
    out_ref[...] = jnp.dot(masks, w_ref[...],
                           preferred_element_type=jnp.float32)


def kernel(white_piece_idx, black_piece_idx, white_castle_idx,
           black_castle_idx, white_ep_idx, black_ep_idx, white_fifty_idx,
           black_fifty_idx, W_white_piece, W_black_piece, W_white_castle,
           W_black_castle, W_white_ep, W_black_ep, W_white_fifty,
           W_black_fifty):
    B = white_piece_idx.shape[0]

    Ww = jnp.transpose(W_white_piece, (1, 0, 2)).reshape(768, _D)
    Wb = jnp.transpose(W_black_piece, (1, 0, 2)).reshape(768, _D)
    Wcat = jnp.concatenate(
        [Ww, Wb, W_white_castle, W_black_castle, W_white_ep, W_black_ep,
         W_white_fifty, W_black_fifty,
         jnp.zeros((4, _D), W_white_piece.dtype)], axis=0)
    Wcat = Wcat.astype(jnp.bfloat16)

    def rep(x, n):
        return jnp.broadcast_to(x[:, None], (x.shape[0], n))

    sm = jnp.concatenate(
        [rep(white_castle_idx, 4), rep(black_castle_idx, 4),
         rep(white_ep_idx, 8), rep(black_ep_idx, 8),
         rep(white_fifty_idx, 2), rep(black_fifty_idx, 2),
         rep(white_fifty_idx, 4)], axis=1).astype(jnp.int32)

    return pl.pallas_call(
        _body,
        grid=(B // _TB,),
        in_specs=[
            pl.BlockSpec((_TB, 64), lambda i: (i, 0)),
            pl.BlockSpec((_TB, 64), lambda i: (i, 0)),
            pl.BlockSpec((_TB, 32), lambda i: (i, 0)),
            pl.BlockSpec((_K, _D), lambda i: (0, 0)),
        ],
        out_specs=pl.BlockSpec((_TB, _D), lambda i: (i, 0)),
        out_shape=jax.ShapeDtypeStruct((B, _D), jnp.float32),
        compiler_params=pltpu.CompilerParams(
            dimension_semantics=("arbitrary",)),
    )(white_piece_idx.astype(jnp.int32), black_piece_idx.astype(jnp.int32),
      sm, Wcat)
